# baseline (speedup 1.0000x reference)
import functools

import jax
import jax.numpy as jnp
from jax import lax
from jax.experimental import pallas as pl
from jax.experimental.pallas import tpu as pltpu
from jax.experimental.pallas import tpu_sc as plsc

N = 10000
E = 320000
D = 128
NUM_ATOM = 100
L_LAYERS = 4
K_HOPS = 3

NC = 2
NS = 16
NW = NC * NS
EW = E // NW
CHUNK = 128
EWP = 10240
NCH = EWP // CHUNK
EPAD = EWP - EW
NP = 10240
RPS = NP // NS


@functools.cache
def _sc_kernels():
    mesh = plsc.VectorSubcoreMesh(
        core_axis_name="c", subcore_axis_name="s",
        num_cores=NC, num_subcores=NS,
    )
    sc_degree = pl.kernel(
        _sc_degree_body,
        out_type=jax.ShapeDtypeStruct((NC, NP, D), jnp.float32),
        mesh=mesh,
        scratch_types=[
            pltpu.VMEM((CHUNK,), jnp.int32),
            pltpu.VMEM((CHUNK,), jnp.int32),
            pltpu.VMEM((CHUNK, D), jnp.float32),
            pltpu.VMEM_SHARED((NP, D), jnp.float32),
            pltpu.SemaphoreType.DMA,
            pltpu.SemaphoreType.DMA,
        ],
    )
    sc_round = pl.kernel(
        _sc_round_body,
        out_type=jax.ShapeDtypeStruct((NC, NP, D), jnp.float32),
        mesh=mesh,
        scratch_types=[
            pltpu.VMEM((CHUNK,), jnp.int32),
            pltpu.VMEM((CHUNK,), jnp.int32),
            pltpu.VMEM((CHUNK,), jnp.int32),
            pltpu.VMEM((CHUNK,), jnp.int32),
            pltpu.VMEM((CHUNK, D), jnp.float32),
            pltpu.VMEM((CHUNK, D), jnp.float32),
            pltpu.VMEM_SHARED((NP, D), jnp.float32),
            pltpu.SemaphoreType.DMA,
            pltpu.SemaphoreType.DMA,
            pltpu.SemaphoreType.DMA,
            pltpu.SemaphoreType.DMA,
        ],
    )
    return sc_degree, sc_round


def _sc_degree_body(dst_hbm, ones_hbm, zeros_hbm, out_hbm,
                    id0, id1, ones_v, acc, ss0, ss1):
    c = lax.axis_index("c")
    s = lax.axis_index("s")
    w = c * NS + s
    ID = (id0, id1)
    SS = (ss0, ss1)
    pltpu.sync_copy(ones_hbm, ones_v)
    pltpu.sync_copy(zeros_hbm, acc.at[pl.ds(s * RPS, RPS)])
    plsc.subcore_barrier()
    def body(i, carry):
        pltpu.sync_copy(dst_hbm.at[w, i], id0)
        pltpu.sync_copy(ones_v, acc.at[id0], add=True)
        return carry

    lax.fori_loop(0, NCH, body, 0)
    plsc.subcore_barrier()
    pltpu.sync_copy(acc.at[pl.ds(s * RPS, RPS)], out_hbm.at[c, pl.ds(s * RPS, RPS)])


def _sc_round_body(xhat_hbm, src_hbm, dst_hbm, zeros_hbm, out_hbm,
                   is0, is1, id0, id1, r0, r1, acc, gs0, gs1, ss0, ss1):
    c = lax.axis_index("c")
    s = lax.axis_index("s")
    w = c * NS + s
    IS = (is0, is1)
    ID = (id0, id1)
    R = (r0, r1)
    GS = (gs0, gs1)
    SS = (ss0, ss1)
    pltpu.sync_copy(zeros_hbm, acc.at[pl.ds(s * RPS, RPS)])
    plsc.subcore_barrier()
    def body(i, carry):
        pltpu.sync_copy(src_hbm.at[w, i], is0)
        pltpu.sync_copy(dst_hbm.at[w, i], id0)
        pltpu.async_copy(xhat_hbm.at[is0], r0, gs0).wait()
        pltpu.sync_copy(r0, acc.at[id0], add=True)
        return carry

    lax.fori_loop(0, NCH, body, 0)
    plsc.subcore_barrier()
    pltpu.sync_copy(acc.at[pl.ds(s * RPS, RPS)], out_hbm.at[c, pl.ds(s * RPS, RPS)])


def _tc_init_body(h_ref, emb_ref, degp_ref, w0_ref, gates_ref,
                  x_ref, isr_ref, xhat_ref, oacc_ref):
    h = h_ref[...]
    atoms = lax.broadcasted_iota(jnp.int32, (1, NUM_ATOM), 1)
    oh = (h == atoms).astype(jnp.float32)
    x = jnp.dot(oh, emb_ref[...], preferred_element_type=jnp.float32, precision=lax.Precision.HIGHEST)
    deg = degp_ref[0, :N, 0:1] + degp_ref[1, :N, 0:1]
    isr = 1.0 / jnp.sqrt(jnp.maximum(deg, 1.0))
    isr_b = jnp.broadcast_to(isr, (N, D))
    g = jax.nn.sigmoid(gates_ref[0, 0])
    x_ref[...] = x
    isr_ref[...] = isr_b
    xhat_ref[...] = x * isr_b
    oacc_ref[...] = g * jnp.dot(x, w0_ref[...], preferred_element_type=jnp.float32, precision=lax.Precision.HIGHEST)


def _tc_hop_body(l, k, parts_ref, isr_ref, w_ref, gates_ref, oacc_ref,
                 oacc_out_ref, xhat_out_ref):
    isr = isr_ref[...]
    xs = isr * (parts_ref[0, :N] + parts_ref[1, :N])
    g = jax.nn.sigmoid(gates_ref[l, k])
    oacc_out_ref[...] = oacc_ref[...] + g * jnp.dot(
        xs, w_ref[...], preferred_element_type=jnp.float32, precision=lax.Precision.HIGHEST)
    xhat_out_ref[...] = xs * isr


def _layer_tail(oacc_ref, hin_ref, snorm_ref, bns_ref, bnb_ref):
    out = oacc_ref[...] * snorm_ref[...]
    mu = jnp.mean(out, axis=0, keepdims=True)
    var = jnp.mean((out - mu) * (out - mu), axis=0, keepdims=True)
    out = (out - mu) / jnp.sqrt(var + 1e-5) * bns_ref[...] + bnb_ref[...]
    out = jnp.maximum(out, 0.0)
    return hin_ref[...] + out


def _tc_tail_body(l, isr_ref, gates_ref, oacc_ref, hin_ref,
                  snorm_ref, bns_ref, bnb_ref, wnext_ref,
                  xnew_ref, oaccn_ref, xhatn_ref):
    x_new = _layer_tail(oacc_ref, hin_ref, snorm_ref, bns_ref, bnb_ref)
    gn = jax.nn.sigmoid(gates_ref[l + 1, 0])
    xnew_ref[...] = x_new
    oaccn_ref[...] = gn * jnp.dot(x_new, wnext_ref[...],
                                  preferred_element_type=jnp.float32, precision=lax.Precision.HIGHEST)
    xhatn_ref[...] = x_new * isr_ref[...]


def _tc_final_body(oacc_ref, hin_ref, snorm_ref, bns_ref, bnb_ref,
                   w1_ref, b1_ref, w2_ref, b2_ref, w3_ref, b3_ref, y_ref):
    x_new = _layer_tail(oacc_ref, hin_ref, snorm_ref, bns_ref, bnb_ref)
    hg = jnp.mean(x_new, axis=0, keepdims=True)
    y = jnp.dot(hg, w1_ref[...], preferred_element_type=jnp.float32, precision=lax.Precision.HIGHEST) + b1_ref[...]
    y = jnp.maximum(y, 0.0)
    y = jnp.dot(y, w2_ref[...], preferred_element_type=jnp.float32, precision=lax.Precision.HIGHEST) + b2_ref[...]
    y = jnp.maximum(y, 0.0)
    y_ref[...] = jnp.dot(y, w3_ref[...], preferred_element_type=jnp.float32, precision=lax.Precision.HIGHEST) + b3_ref[...]


def _f32(shape):
    return jax.ShapeDtypeStruct(shape, jnp.float32)


def kernel(h, edge_index, e, snorm_n, snorm_e, emb, W, gates, bn_scale,
           bn_bias, w1, b1, w2, b2, w3, b3):
    src = jnp.pad(edge_index[0].astype(jnp.int32).reshape(NW, EW),
                  ((0, 0), (0, EPAD))).reshape(NW, NCH, CHUNK)
    dst = jnp.pad(edge_index[1].astype(jnp.int32).reshape(NW, EW),
                  ((0, 0), (0, EPAD)), constant_values=N).reshape(NW, NCH, CHUNK)
    onesD = jnp.ones((CHUNK, D), jnp.float32)
    zerosD = jnp.zeros((RPS, D), jnp.float32)
    h2 = h.astype(jnp.int32).reshape(N, 1)

    sc_degree, sc_round = _sc_kernels()
    degp = sc_degree(dst, onesD, zerosD)

    x, isr, xhat, oacc = pl.pallas_call(
        _tc_init_body,
        out_shape=[_f32((N, D))] * 4,
    )(h2, emb, degp, W[0, 0], gates)

    y = None
    for l in range(L_LAYERS):
        for k in range(1, K_HOPS + 1):
            parts = sc_round(xhat, src, dst, zerosD)
            oacc, xhat = pl.pallas_call(
                functools.partial(_tc_hop_body, l, k),
                out_shape=[_f32((N, D))] * 2,
            )(parts, isr, W[l, k], gates, oacc)
        if l < L_LAYERS - 1:
            x, oacc, xhat = pl.pallas_call(
                functools.partial(_tc_tail_body, l),
                out_shape=[_f32((N, D))] * 3,
            )(isr, gates, oacc, x, snorm_n,
              bn_scale[l].reshape(1, D), bn_bias[l].reshape(1, D),
              W[l + 1, 0])
        else:
            y = pl.pallas_call(
                _tc_final_body,
                out_shape=_f32((1, 1)),
            )(oacc, x, snorm_n,
              bn_scale[l].reshape(1, D), bn_bias[l].reshape(1, D),
              w1, b1.reshape(1, D // 2), w2, b2.reshape(1, D // 4),
              w3, b3.reshape(1, 1))
    return y


# speedup vs baseline: 1.7605x; 1.7605x over previous
import functools

import jax
import jax.numpy as jnp
from jax import lax
from jax.experimental import pallas as pl
from jax.experimental.pallas import tpu as pltpu
from jax.experimental.pallas import tpu_sc as plsc

N = 10000
E = 320000
D = 128
NUM_ATOM = 100
L_LAYERS = 4
K_HOPS = 3

NC = 2
NS = 16
NW = NC * NS
EW = E // NW
CHUNK = 80
EWP = 10080
NCH = EWP // CHUNK
EPAD = EWP - EW
NP = 10240
RPS = NP // NS


@functools.cache
def _sc_kernels():
    mesh = plsc.VectorSubcoreMesh(
        core_axis_name="c", subcore_axis_name="s",
        num_cores=NC, num_subcores=NS,
    )
    sc_degree = pl.kernel(
        _sc_degree_body,
        out_type=jax.ShapeDtypeStruct((NC, NP, D), jnp.float32),
        mesh=mesh,
        scratch_types=[
            pltpu.VMEM((CHUNK,), jnp.int32),
            pltpu.VMEM((CHUNK,), jnp.int32),
            pltpu.VMEM((CHUNK, D), jnp.float32),
            pltpu.VMEM_SHARED((NP, D), jnp.float32),
            pltpu.SemaphoreType.DMA,
            pltpu.SemaphoreType.DMA,
        ],
    )
    sc_round = pl.kernel(
        _sc_round_body,
        out_type=jax.ShapeDtypeStruct((NC, NP, D), jnp.float32),
        mesh=mesh,
        scratch_types=[
            pltpu.VMEM((CHUNK,), jnp.int32),
            pltpu.VMEM((CHUNK,), jnp.int32),
            pltpu.VMEM((CHUNK,), jnp.int32),
            pltpu.VMEM((CHUNK,), jnp.int32),
            pltpu.VMEM((CHUNK, D), jnp.float32),
            pltpu.VMEM((CHUNK, D), jnp.float32),
            pltpu.VMEM_SHARED((NP, D), jnp.float32),
            pltpu.SemaphoreType.DMA,
            pltpu.SemaphoreType.DMA,
            pltpu.SemaphoreType.DMA,
            pltpu.SemaphoreType.DMA,
        ],
    )
    return sc_degree, sc_round


def _sc_degree_body(dst_hbm, ones_hbm, zeros_hbm, out_hbm,
                    id0, id1, ones_v, acc, ss0, ss1):
    c = lax.axis_index("c")
    s = lax.axis_index("s")
    w = c * NS + s
    ID = (id0, id1)
    SS = (ss0, ss1)
    pltpu.sync_copy(ones_hbm, ones_v)
    pltpu.sync_copy(zeros_hbm, acc.at[pl.ds(s * RPS, RPS)])
    plsc.subcore_barrier()
    def body(i, carry):
        pltpu.sync_copy(dst_hbm.at[w, i], id0)
        pltpu.sync_copy(ones_v, acc.at[id0], add=True)
        return carry

    lax.fori_loop(0, NCH, body, 0)
    plsc.subcore_barrier()
    pltpu.sync_copy(acc.at[pl.ds(s * RPS, RPS)], out_hbm.at[c, pl.ds(s * RPS, RPS)])


def _sc_round_body(xhat_hbm, src_hbm, dst_hbm, zeros_hbm, out_hbm,
                   is0, is1, id0, id1, r0, r1, acc, gs0, gs1, ss0, ss1):
    c = lax.axis_index("c")
    s = lax.axis_index("s")
    w = c * NS + s
    IS = (is0, is1)
    ID = (id0, id1)
    R = (r0, r1)
    GS = (gs0, gs1)
    SS = (ss0, ss1)
    pltpu.sync_copy(zeros_hbm, acc.at[pl.ds(s * RPS, RPS)])
    plsc.subcore_barrier()
    pltpu.sync_copy(src_hbm.at[w, 0], is0)
    pltpu.sync_copy(dst_hbm.at[w, 0], id0)
    pltpu.async_copy(xhat_hbm.at[is0], r0, gs0)
    pltpu.sync_copy(src_hbm.at[w, 1], is1)
    pltpu.sync_copy(dst_hbm.at[w, 1], id1)
    pltpu.async_copy(xhat_hbm.at[is1], r1, gs1)
    pltpu.make_async_copy(xhat_hbm.at[is0], r0, gs0).wait()
    pltpu.async_copy(r0, acc.at[id0], ss0, add=True)

    def outer(j, carry):
        for b in range(2):
            cidx = 2 * j + 1 + b
            v = 1 - b
            nv = b
            pltpu.make_async_copy(R[nv], acc.at[ID[nv]], SS[nv]).wait()
            pltpu.sync_copy(src_hbm.at[w, cidx + 1], IS[nv])
            pltpu.sync_copy(dst_hbm.at[w, cidx + 1], ID[nv])
            pltpu.async_copy(xhat_hbm.at[IS[nv]], R[nv], GS[nv])
            pltpu.make_async_copy(xhat_hbm.at[IS[v]], R[v], GS[v]).wait()
            pltpu.async_copy(R[v], acc.at[ID[v]], SS[v], add=True)
        return carry

    lax.fori_loop(0, (NCH - 2) // 2, outer, 0)
    pltpu.make_async_copy(R[0], acc.at[ID[0]], SS[0]).wait()
    pltpu.make_async_copy(xhat_hbm.at[is1], r1, gs1).wait()
    pltpu.async_copy(r1, acc.at[id1], ss1, add=True)
    pltpu.make_async_copy(R[1], acc.at[ID[1]], SS[1]).wait()
    plsc.subcore_barrier()
    pltpu.sync_copy(acc.at[pl.ds(s * RPS, RPS)], out_hbm.at[c, pl.ds(s * RPS, RPS)])


def _tc_init_body(h_ref, emb_ref, degp_ref, w0_ref, gates_ref,
                  x_ref, isr_ref, xhat_ref, oacc_ref):
    h = h_ref[...]
    atoms = lax.broadcasted_iota(jnp.int32, (1, NUM_ATOM), 1)
    oh = (h == atoms).astype(jnp.float32)
    x = jnp.dot(oh, emb_ref[...], preferred_element_type=jnp.float32, precision=lax.Precision.HIGHEST)
    deg = degp_ref[0, :N, 0:1] + degp_ref[1, :N, 0:1]
    isr = 1.0 / jnp.sqrt(jnp.maximum(deg, 1.0))
    isr_b = jnp.broadcast_to(isr, (N, D))
    g = jax.nn.sigmoid(gates_ref[0, 0])
    x_ref[...] = x
    isr_ref[...] = isr_b
    xhat_ref[...] = x * isr_b
    oacc_ref[...] = g * jnp.dot(x, w0_ref[...], preferred_element_type=jnp.float32, precision=lax.Precision.HIGHEST)


def _tc_hop_body(l, k, parts_ref, isr_ref, w_ref, gates_ref, oacc_ref,
                 oacc_out_ref, xhat_out_ref):
    isr = isr_ref[...]
    xs = isr * (parts_ref[0, :N] + parts_ref[1, :N])
    g = jax.nn.sigmoid(gates_ref[l, k])
    oacc_out_ref[...] = oacc_ref[...] + g * jnp.dot(
        xs, w_ref[...], preferred_element_type=jnp.float32, precision=lax.Precision.HIGHEST)
    xhat_out_ref[...] = xs * isr


def _layer_tail(oacc_ref, hin_ref, snorm_ref, bns_ref, bnb_ref):
    out = oacc_ref[...] * snorm_ref[...]
    mu = jnp.mean(out, axis=0, keepdims=True)
    var = jnp.mean((out - mu) * (out - mu), axis=0, keepdims=True)
    out = (out - mu) / jnp.sqrt(var + 1e-5) * bns_ref[...] + bnb_ref[...]
    out = jnp.maximum(out, 0.0)
    return hin_ref[...] + out


def _tc_tail_body(l, isr_ref, gates_ref, oacc_ref, hin_ref,
                  snorm_ref, bns_ref, bnb_ref, wnext_ref,
                  xnew_ref, oaccn_ref, xhatn_ref):
    x_new = _layer_tail(oacc_ref, hin_ref, snorm_ref, bns_ref, bnb_ref)
    gn = jax.nn.sigmoid(gates_ref[l + 1, 0])
    xnew_ref[...] = x_new
    oaccn_ref[...] = gn * jnp.dot(x_new, wnext_ref[...],
                                  preferred_element_type=jnp.float32, precision=lax.Precision.HIGHEST)
    xhatn_ref[...] = x_new * isr_ref[...]


def _tc_final_body(oacc_ref, hin_ref, snorm_ref, bns_ref, bnb_ref,
                   w1_ref, b1_ref, w2_ref, b2_ref, w3_ref, b3_ref, y_ref):
    x_new = _layer_tail(oacc_ref, hin_ref, snorm_ref, bns_ref, bnb_ref)
    hg = jnp.mean(x_new, axis=0, keepdims=True)
    y = jnp.dot(hg, w1_ref[...], preferred_element_type=jnp.float32, precision=lax.Precision.HIGHEST) + b1_ref[...]
    y = jnp.maximum(y, 0.0)
    y = jnp.dot(y, w2_ref[...], preferred_element_type=jnp.float32, precision=lax.Precision.HIGHEST) + b2_ref[...]
    y = jnp.maximum(y, 0.0)
    y_ref[...] = jnp.dot(y, w3_ref[...], preferred_element_type=jnp.float32, precision=lax.Precision.HIGHEST) + b3_ref[...]


def _f32(shape):
    return jax.ShapeDtypeStruct(shape, jnp.float32)


def kernel(h, edge_index, e, snorm_n, snorm_e, emb, W, gates, bn_scale,
           bn_bias, w1, b1, w2, b2, w3, b3):
    src = jnp.pad(edge_index[0].astype(jnp.int32).reshape(NW, EW),
                  ((0, 0), (0, EPAD))).reshape(NW, NCH, CHUNK)
    dst = jnp.pad(edge_index[1].astype(jnp.int32).reshape(NW, EW),
                  ((0, 0), (0, EPAD)), constant_values=N).reshape(NW, NCH, CHUNK)
    onesD = jnp.ones((CHUNK, D), jnp.float32)
    zerosD = jnp.zeros((RPS, D), jnp.float32)
    h2 = h.astype(jnp.int32).reshape(N, 1)

    sc_degree, sc_round = _sc_kernels()
    degp = sc_degree(dst, onesD, zerosD)

    x, isr, xhat, oacc = pl.pallas_call(
        _tc_init_body,
        out_shape=[_f32((N, D))] * 4,
    )(h2, emb, degp, W[0, 0], gates)

    y = None
    for l in range(L_LAYERS):
        for k in range(1, K_HOPS + 1):
            parts = sc_round(xhat, src, dst, zerosD)
            oacc, xhat = pl.pallas_call(
                functools.partial(_tc_hop_body, l, k),
                out_shape=[_f32((N, D))] * 2,
            )(parts, isr, W[l, k], gates, oacc)
        if l < L_LAYERS - 1:
            x, oacc, xhat = pl.pallas_call(
                functools.partial(_tc_tail_body, l),
                out_shape=[_f32((N, D))] * 3,
            )(isr, gates, oacc, x, snorm_n,
              bn_scale[l].reshape(1, D), bn_bias[l].reshape(1, D),
              W[l + 1, 0])
        else:
            y = pl.pallas_call(
                _tc_final_body,
                out_shape=_f32((1, 1)),
            )(oacc, x, snorm_n,
              bn_scale[l].reshape(1, D), bn_bias[l].reshape(1, D),
              w1, b1.reshape(1, D // 2), w2, b2.reshape(1, D // 4),
              w3, b3.reshape(1, 1))
    return y
